# baseline (speedup 1.0000x reference)
import functools

import jax
import jax.numpy as jnp
from jax import lax
from jax.experimental import pallas as pl
from jax.experimental.pallas import tpu as pltpu
from jax.experimental.pallas import tpu_sc as plsc

VOCAB = 1000000
EMBED_DIM = 4
BATCH = 16384
SEQ = 200

NC = 2
NS = 16
NW = NC * NS

DP = 8
N_TOTAL = BATCH * SEQ
PER_TILE = N_TOTAL // NW
BLK = 4096
N_BLK = PER_TILE // BLK


def _gather_body(table_hbm, ids_hbm, out_hbm,
                 idx0, idx1, rows0, rows1,
                 sem_i0, sem_i1, sem_r0, sem_r1, sem_o0, sem_o1):
    wid = lax.axis_index("s") * NC + lax.axis_index("c")
    idx_bufs = [idx0, idx1]
    rows_bufs = [rows0, rows1]
    sem_i = [sem_i0, sem_i1]
    sem_r = [sem_r0, sem_r1]
    sem_o = [sem_o0, sem_o1]

    idx_copies = [None] * (N_BLK + 2)
    row_copies = [None] * (N_BLK + 1)
    out_copies = [None] * N_BLK

    pltpu.sync_copy(ids_hbm.at[wid, pl.ds(0, BLK)], idx_bufs[0])
    row_copies[0] = pltpu.async_copy(
        table_hbm.at[idx_bufs[0]], rows_bufs[0], sem_r[0])
    if N_BLK > 1:
        idx_copies[1] = pltpu.async_copy(
            ids_hbm.at[wid, pl.ds(BLK, BLK)], idx_bufs[1], sem_i[1])

    for g in range(N_BLK):
        p, q = g % 2, (g + 1) % 2
        if g + 1 < N_BLK:
            idx_copies[g + 1].wait()
            if g >= 1:
                out_copies[g - 1].wait()
            row_copies[g + 1] = pltpu.async_copy(
                table_hbm.at[idx_bufs[q]], rows_bufs[q], sem_r[q])
        row_copies[g].wait()
        if g + 2 < N_BLK:
            idx_copies[g + 2] = pltpu.async_copy(
                ids_hbm.at[wid, pl.ds((g + 2) * BLK, BLK)],
                idx_bufs[p], sem_i[p])
        out_copies[g] = pltpu.async_copy(
            rows_bufs[p].at[:, pl.ds(0, EMBED_DIM)],
            out_hbm.at[wid, pl.ds(g * BLK, BLK)], sem_o[p])

    out_copies[N_BLK - 1].wait()
    if N_BLK > 1:
        out_copies[N_BLK - 2].wait()


@jax.jit
def _embed_gather(ids_flat, table_pad):
    mesh = plsc.VectorSubcoreMesh(core_axis_name="c", subcore_axis_name="s",
                                  num_cores=NC, num_subcores=NS)
    f = pl.kernel(
        _gather_body,
        out_type=jax.ShapeDtypeStruct((NW, PER_TILE, EMBED_DIM), jnp.float32),
        mesh=mesh,
        scratch_types=[
            pltpu.VMEM((BLK,), jnp.int32),
            pltpu.VMEM((BLK,), jnp.int32),
            pltpu.VMEM((BLK, DP), jnp.float32),
            pltpu.VMEM((BLK, DP), jnp.float32),
            pltpu.SemaphoreType.DMA,
            pltpu.SemaphoreType.DMA,
            pltpu.SemaphoreType.DMA,
            pltpu.SemaphoreType.DMA,
            pltpu.SemaphoreType.DMA,
            pltpu.SemaphoreType.DMA,
        ],
        compiler_params=pltpu.CompilerParams(use_tc_tiling_on_sc=False,
                                             needs_layout_passes=False),
    )
    return f(table_pad, ids_flat)


def kernel(input_ids, embed_weight):
    ids = input_ids.astype(jnp.int32).reshape(NW, PER_TILE)
    table_pad = jnp.pad(embed_weight, ((0, 0), (0, DP - EMBED_DIM)))
    out = _embed_gather(ids, table_pad)
    return out.reshape(BATCH, SEQ, EMBED_DIM)


# speedup vs baseline: 2.2907x; 2.2907x over previous
import functools

import jax
import jax.numpy as jnp
from jax import lax
from jax.experimental import pallas as pl
from jax.experimental.pallas import tpu as pltpu
from jax.experimental.pallas import tpu_sc as plsc

VOCAB = 1000000
EMBED_DIM = 4
BATCH = 16384
SEQ = 200

NC = 2
NS = 16
NW = NC * NS

DP = 8
N_TOTAL = BATCH * SEQ
PER_TILE = N_TOTAL // NW
BLK = 2048
N_BLK = PER_TILE // BLK
OUT_BLK = BLK * EMBED_DIM


def _gather_body(table_hbm, ids_hbm, out_hbm,
                 idx0, idx1, rows0, rows1, out0, out1,
                 sem_i0, sem_i1, sem_r0, sem_r1, sem_o0, sem_o1):
    wid = lax.axis_index("s") * NC + lax.axis_index("c")
    idx_bufs = [idx0, idx1]
    rows_bufs = [rows0, rows1]
    out_bufs = [out0, out1]
    sem_i = [sem_i0, sem_i1]
    sem_r = [sem_r0, sem_r1]
    sem_o = [sem_o0, sem_o1]

    iota = lax.iota(jnp.int32, 16)
    row_pat = jnp.right_shift(iota, 2)
    col_pat = jnp.bitwise_and(iota, 3)

    idx_copies = [None] * (N_BLK + 2)
    row_copies = [None] * (N_BLK + 1)
    out_copies = [None] * N_BLK

    def compress(rows_ref, out_ref):
        def step(j, _):
            vals = plsc.load_gather(rows_ref, [4 * j + row_pat, col_pat])
            out_ref[pl.ds(16 * j, 16)] = vals
            return _
        lax.fori_loop(0, OUT_BLK // 16, step, 0)

    pltpu.sync_copy(ids_hbm.at[wid, pl.ds(0, BLK)], idx_bufs[0])
    row_copies[0] = pltpu.async_copy(
        table_hbm.at[idx_bufs[0]], rows_bufs[0], sem_r[0])
    idx_copies[1] = pltpu.async_copy(
        ids_hbm.at[wid, pl.ds(BLK, BLK)], idx_bufs[1], sem_i[1])

    for g in range(N_BLK):
        p, q = g % 2, (g + 1) % 2
        if g + 1 < N_BLK:
            idx_copies[g + 1].wait()
            row_copies[g + 1] = pltpu.async_copy(
                table_hbm.at[idx_bufs[q]], rows_bufs[q], sem_r[q])
        row_copies[g].wait()
        if g + 2 < N_BLK:
            idx_copies[g + 2] = pltpu.async_copy(
                ids_hbm.at[wid, pl.ds((g + 2) * BLK, BLK)],
                idx_bufs[p], sem_i[p])
        if g >= 2:
            out_copies[g - 2].wait()
        compress(rows_bufs[p], out_bufs[p])
        out_copies[g] = pltpu.async_copy(
            out_bufs[p], out_hbm.at[wid, pl.ds(g * OUT_BLK, OUT_BLK)],
            sem_o[p])

    out_copies[N_BLK - 1].wait()
    out_copies[N_BLK - 2].wait()


@jax.jit
def _embed_gather(ids_flat, table_pad):
    mesh = plsc.VectorSubcoreMesh(core_axis_name="c", subcore_axis_name="s",
                                  num_cores=NC, num_subcores=NS)
    f = pl.kernel(
        _gather_body,
        out_type=jax.ShapeDtypeStruct((NW, PER_TILE * EMBED_DIM), jnp.float32),
        mesh=mesh,
        scratch_types=[
            pltpu.VMEM((BLK,), jnp.int32),
            pltpu.VMEM((BLK,), jnp.int32),
            pltpu.VMEM((BLK, DP), jnp.float32),
            pltpu.VMEM((BLK, DP), jnp.float32),
            pltpu.VMEM((OUT_BLK,), jnp.float32),
            pltpu.VMEM((OUT_BLK,), jnp.float32),
            pltpu.SemaphoreType.DMA,
            pltpu.SemaphoreType.DMA,
            pltpu.SemaphoreType.DMA,
            pltpu.SemaphoreType.DMA,
            pltpu.SemaphoreType.DMA,
            pltpu.SemaphoreType.DMA,
        ],
        compiler_params=pltpu.CompilerParams(use_tc_tiling_on_sc=False,
                                             needs_layout_passes=False),
    )
    return f(table_pad, ids_flat)


def kernel(input_ids, embed_weight):
    ids = input_ids.astype(jnp.int32).reshape(NW, PER_TILE)
    table_pad = jnp.pad(embed_weight, ((0, 0), (0, DP - EMBED_DIM)))
    out = _embed_gather(ids, table_pad)
    return out.reshape(BATCH, SEQ, EMBED_DIM)


# speedup vs baseline: 2.3214x; 1.0134x over previous
import functools

import jax
import jax.numpy as jnp
from jax import lax
from jax.experimental import pallas as pl
from jax.experimental.pallas import tpu as pltpu
from jax.experimental.pallas import tpu_sc as plsc

VOCAB = 1000000
EMBED_DIM = 4
BATCH = 16384
SEQ = 200

NC = 2
NS = 16
NW = NC * NS

DP = 8
N_TOTAL = BATCH * SEQ
PER_TILE = N_TOTAL // NW
BLK = 2048
N_BLK = PER_TILE // BLK
OUT_BLK = BLK * EMBED_DIM


def _gather_body(table_hbm, ids_hbm, out_hbm,
                 idx0, idx1, idx2, half0, half1, rows0, rows1, out0, out1,
                 sem_i0, sem_i1, sem_i2, sem_r0, sem_r1, sem_o0, sem_o1):
    wid = lax.axis_index("s") * NC + lax.axis_index("c")
    idx_bufs = [idx0, idx1, idx2]
    half_bufs = [half0, half1]
    rows_bufs = [rows0, rows1]
    out_bufs = [out0, out1]
    sem_i = [sem_i0, sem_i1, sem_i2]
    sem_r = [sem_r0, sem_r1]
    sem_o = [sem_o0, sem_o1]

    iota = lax.iota(jnp.int32, 16)
    row_pat = jnp.right_shift(iota, 2)
    col_pat = jnp.bitwise_and(iota, 3)

    idx_copies = [None] * (N_BLK + 2)
    row_copies = [None] * (N_BLK + 1)
    out_copies = [None] * N_BLK

    def halve(idx_ref, half_ref):
        def step(j, _):
            v = idx_ref[pl.ds(16 * j, 16)]
            half_ref[pl.ds(16 * j, 16)] = jnp.right_shift(v, 1)
            return _
        lax.fori_loop(0, BLK // 16, step, 0)

    def compress(idx_ref, rows_ref, out_ref):
        def step(j, _):
            rvec = 4 * j + row_pat
            par = jnp.bitwise_and(plsc.load_gather(idx_ref, [rvec]), 1)
            cvec = col_pat + jnp.left_shift(par, 2)
            out_ref[pl.ds(16 * j, 16)] = plsc.load_gather(
                rows_ref, [rvec, cvec])
            return _
        lax.fori_loop(0, OUT_BLK // 16, step, 0)

    pltpu.sync_copy(ids_hbm.at[wid, pl.ds(0, BLK)], idx_bufs[0])
    halve(idx_bufs[0], half_bufs[0])
    row_copies[0] = pltpu.async_copy(
        table_hbm.at[half_bufs[0]], rows_bufs[0], sem_r[0])
    idx_copies[1] = pltpu.async_copy(
        ids_hbm.at[wid, pl.ds(BLK, BLK)], idx_bufs[1], sem_i[1])

    for g in range(N_BLK):
        p, q = g % 2, (g + 1) % 2
        i_cur, i_nxt, i_pre = g % 3, (g + 1) % 3, (g + 2) % 3
        if g + 1 < N_BLK:
            idx_copies[g + 1].wait()
            halve(idx_bufs[i_nxt], half_bufs[q])
            row_copies[g + 1] = pltpu.async_copy(
                table_hbm.at[half_bufs[q]], rows_bufs[q], sem_r[q])
        row_copies[g].wait()
        if g + 2 < N_BLK:
            idx_copies[g + 2] = pltpu.async_copy(
                ids_hbm.at[wid, pl.ds((g + 2) * BLK, BLK)],
                idx_bufs[i_pre], sem_i[i_pre])
        if g >= 2:
            out_copies[g - 2].wait()
        compress(idx_bufs[i_cur], rows_bufs[p], out_bufs[p])
        out_copies[g] = pltpu.async_copy(
            out_bufs[p], out_hbm.at[wid, pl.ds(g * OUT_BLK, OUT_BLK)],
            sem_o[p])

    out_copies[N_BLK - 1].wait()
    out_copies[N_BLK - 2].wait()


@jax.jit
def _embed_gather(ids_flat, table_pairs):
    mesh = plsc.VectorSubcoreMesh(core_axis_name="c", subcore_axis_name="s",
                                  num_cores=NC, num_subcores=NS)
    f = pl.kernel(
        _gather_body,
        out_type=jax.ShapeDtypeStruct((NW, PER_TILE * EMBED_DIM), jnp.float32),
        mesh=mesh,
        scratch_types=[
            pltpu.VMEM((BLK,), jnp.int32),
            pltpu.VMEM((BLK,), jnp.int32),
            pltpu.VMEM((BLK,), jnp.int32),
            pltpu.VMEM((BLK,), jnp.int32),
            pltpu.VMEM((BLK,), jnp.int32),
            pltpu.VMEM((BLK, DP), jnp.float32),
            pltpu.VMEM((BLK, DP), jnp.float32),
            pltpu.VMEM((OUT_BLK,), jnp.float32),
            pltpu.VMEM((OUT_BLK,), jnp.float32),
            pltpu.SemaphoreType.DMA,
            pltpu.SemaphoreType.DMA,
            pltpu.SemaphoreType.DMA,
            pltpu.SemaphoreType.DMA,
            pltpu.SemaphoreType.DMA,
            pltpu.SemaphoreType.DMA,
            pltpu.SemaphoreType.DMA,
        ],
        compiler_params=pltpu.CompilerParams(use_tc_tiling_on_sc=False,
                                             needs_layout_passes=False),
    )
    return f(table_pairs, ids_flat)


def kernel(input_ids, embed_weight):
    ids = input_ids.astype(jnp.int32).reshape(NW, PER_TILE)
    table_pairs = embed_weight.reshape(VOCAB // 2, 2 * EMBED_DIM)
    out = _embed_gather(ids, table_pairs)
    return out.reshape(BATCH, SEQ, EMBED_DIM)
